# baseline (speedup 1.0000x reference)
import functools

import jax
import jax.numpy as jnp
from jax import lax
from jax.experimental import pallas as pl
from jax.experimental.pallas import tpu as pltpu
from jax.experimental.pallas import tpu_sc as plsc

B_ROWS = 16384
SEQ = 200
D = 128
B_TOTAL = B_ROWS * SEQ
NUM_WORKERS = 32
B_PER_W = B_TOTAL // NUM_WORKERS
CHUNK = 128
N_CHUNKS = B_PER_W // CHUNK


def _sc_body(idx_hbm, table_hbm, out_hbm, idx_v, rows_v, sem):
    c = lax.axis_index("c")
    s = lax.axis_index("s")
    wid = s * 2 + c
    base = wid * B_PER_W

    def chunk_body(j, carry):
        off = base + j * CHUNK
        pltpu.sync_copy(idx_hbm.at[pl.ds(off, CHUNK)], idx_v)
        pltpu.async_copy(table_hbm.at[idx_v], rows_v, sem).wait()
        pltpu.sync_copy(rows_v, out_hbm.at[pl.ds(off, CHUNK)])
        return carry

    lax.fori_loop(0, N_CHUNKS, chunk_body, 0)


def kernel(group_idx, weight):
    idx = group_idx.reshape(B_TOTAL)
    mesh = plsc.VectorSubcoreMesh(core_axis_name="c", subcore_axis_name="s")
    run = functools.partial(
        pl.kernel,
        mesh=mesh,
        out_type=jax.ShapeDtypeStruct((B_TOTAL, D), jnp.float32),
        scratch_types=[
            pltpu.VMEM((CHUNK,), jnp.int32),
            pltpu.VMEM((CHUNK, D), jnp.float32),
            pltpu.SemaphoreType.DMA,
        ],
    )(_sc_body)
    out = run(idx, weight)
    return out.reshape(B_ROWS, SEQ, D)


# speedup vs baseline: 21.3769x; 21.3769x over previous
import functools

import jax
import jax.numpy as jnp
from jax import lax
from jax.experimental import pallas as pl
from jax.experimental.pallas import tpu as pltpu
from jax.experimental.pallas import tpu_sc as plsc

B_ROWS = 16384
SEQ = 200
D = 128
B_TOTAL = B_ROWS * SEQ
NUM_WORKERS = 32
B_PER_W = B_TOTAL // NUM_WORKERS
C = 128
NCH = B_PER_W // C


def _sc_body(idx_hbm, table_hbm, out_hbm, table_v, idx_v, out_v,
             sem_i0, sem_i1, sem_g, sem_o0, sem_o1):
    cid = lax.axis_index("c")
    sid = lax.axis_index("s")
    wid = sid * 2 + cid
    base = wid * B_PER_W
    sem_i = (sem_i0, sem_i1)
    sem_o = (sem_o0, sem_o1)

    @pl.when(sid == 0)
    def _():
        pltpu.sync_copy(table_hbm, table_v)

    plsc.subcore_barrier()

    def idx_copy(j, b):
        return pltpu.make_async_copy(
            idx_hbm.at[pl.ds(base + j * C, C)], idx_v.at[b], sem_i[b])

    def out_copy(j, b):
        return pltpu.make_async_copy(
            out_v.at[b], out_hbm.at[pl.ds(base + j * C, C)], sem_o[b])

    idx_copy(0, 0).start()
    idx_copy(1, 1).start()

    def chunk_pair(j2, carry):
        for b in (0, 1):
            j = j2 * 2 + b
            idx_copy(j, b).wait()

            @pl.when(j2 > 0)
            def _():
                out_copy(j, b).wait()

            pltpu.async_copy(
                table_v.at[idx_v.at[b]], out_v.at[b], sem_g).wait()

            @pl.when(j < NCH - 2)
            def _():
                idx_copy(j + 2, b).start()

            out_copy(j, b).start()
        return carry

    lax.fori_loop(0, NCH // 2, chunk_pair, 0)
    out_copy(NCH - 2, 0).wait()
    out_copy(NCH - 1, 1).wait()


def kernel(group_idx, weight):
    idx = group_idx.reshape(B_TOTAL)
    mesh = plsc.VectorSubcoreMesh(core_axis_name="c", subcore_axis_name="s")
    run = functools.partial(
        pl.kernel,
        mesh=mesh,
        out_type=jax.ShapeDtypeStruct((B_TOTAL, D), jnp.float32),
        scratch_types=[
            pltpu.VMEM_SHARED((17, D), jnp.float32),
            pltpu.VMEM((2, C), jnp.int32),
            pltpu.VMEM((2, C, D), jnp.float32),
            pltpu.SemaphoreType.DMA,
            pltpu.SemaphoreType.DMA,
            pltpu.SemaphoreType.DMA,
            pltpu.SemaphoreType.DMA,
            pltpu.SemaphoreType.DMA,
        ],
    )(_sc_body)
    out = run(idx, weight)
    return out.reshape(B_ROWS, SEQ, D)


# speedup vs baseline: 22.9577x; 1.0740x over previous
import functools

import jax
import jax.numpy as jnp
from jax import lax
from jax.experimental import pallas as pl
from jax.experimental.pallas import tpu as pltpu
from jax.experimental.pallas import tpu_sc as plsc

B_ROWS = 16384
SEQ = 200
D = 128
B_TOTAL = B_ROWS * SEQ
NUM_WORKERS = 32
B_PER_W = B_TOTAL // NUM_WORKERS
C = 128
NCH = B_PER_W // C
NB = 4
NJ4 = NCH // NB


def _sc_body(idx_hbm, table_hbm, out_hbm, table_v, idx_v, out_v, *sems):
    sem_i = sems[0:NB]
    sem_g = sems[NB:2 * NB]
    sem_o = sems[2 * NB:3 * NB]
    cid = lax.axis_index("c")
    sid = lax.axis_index("s")
    wid = sid * 2 + cid
    base = wid * B_PER_W

    @pl.when(sid == 0)
    def _():
        pltpu.sync_copy(table_hbm, table_v)

    plsc.subcore_barrier()

    def idx_copy(j, b):
        return pltpu.make_async_copy(
            idx_hbm.at[pl.ds(base + j * C, C)], idx_v.at[b], sem_i[b])

    def gather_copy(b):
        return pltpu.make_async_copy(
            table_v.at[idx_v.at[b]], out_v.at[b], sem_g[b])

    def out_copy(j, b):
        return pltpu.make_async_copy(
            out_v.at[b], out_hbm.at[pl.ds(base + j * C, C)], sem_o[b])

    for b in range(NB):
        idx_copy(b, b).start()
    idx_copy(0, 0).wait()
    gather_copy(0).start()

    def outer(j4, carry):
        for b in range(NB):
            j = j4 * NB + b
            bn = (b + 1) % NB
            if b < NB - 1:
                idx_copy(j + 1, bn).wait()

                @pl.when(j4 > 0)
                def _():
                    out_copy(j - 3, bn).wait()

                gather_copy(bn).start()
            else:
                @pl.when(j4 < NJ4 - 1)
                def _():
                    idx_copy(j + 1, bn).wait()
                    out_copy(j - 3, bn).wait()
                    gather_copy(bn).start()

            gather_copy(b).wait()
            out_copy(j, b).start()

            @pl.when(j4 < NJ4 - 1)
            def _():
                idx_copy(j + NB, b).start()
        return carry

    lax.fori_loop(0, NJ4, outer, 0)
    for b in range(NB):
        out_copy(NCH - NB + b, b).wait()


def kernel(group_idx, weight):
    idx = group_idx.reshape(B_TOTAL)
    mesh = plsc.VectorSubcoreMesh(core_axis_name="c", subcore_axis_name="s")
    run = functools.partial(
        pl.kernel,
        mesh=mesh,
        out_type=jax.ShapeDtypeStruct((B_TOTAL, D), jnp.float32),
        scratch_types=[
            pltpu.VMEM_SHARED((17, D), jnp.float32),
            pltpu.VMEM((NB, C), jnp.int32),
            pltpu.VMEM((NB, C, D), jnp.float32),
        ] + [pltpu.SemaphoreType.DMA] * (3 * NB),
    )(_sc_body)
    out = run(idx, weight)
    return out.reshape(B_ROWS, SEQ, D)
